# Initial kernel scaffold; baseline (speedup 1.0000x reference)
import jax
import jax.numpy as jnp
from jax.experimental import pallas as pl


def kernel(x, tables):
    raise NotImplementedError("write your pallas kernel here")


# speedup vs baseline: 6.7599x; 6.7599x over previous
import functools

import jax
import jax.numpy as jnp
from jax import lax
from jax.experimental import pallas as pl
from jax.experimental.pallas import tpu as pltpu
from jax.experimental.pallas import tpu_sc as plsc

B, G, C, D = 16384, 26, 100, 32

BB = 256


def _argmax_body(x_ref, o_ref):
    xv = x_ref[...]
    m = jnp.max(xv, axis=-1, keepdims=True)
    iota_c = lax.broadcasted_iota(jnp.int32, xv.shape, 2)
    cand = jnp.where(xv == m, iota_c, C)
    idx = jnp.min(cand, axis=-1)
    goff = lax.broadcasted_iota(jnp.int32, idx.shape, 1) * C
    o_ref[...] = idx + goff


def _tc_argmax(x3):
    return pl.pallas_call(
        _argmax_body,
        grid=(B // BB,),
        in_specs=[pl.BlockSpec((BB, G, C), lambda i: (i, 0, 0))],
        out_specs=pl.BlockSpec((BB, G), lambda i: (i, 0)),
        out_shape=jax.ShapeDtypeStruct((B, G), jnp.int32),
    )(x3)


NC, NS = 2, 16
NW = NC * NS
TOK = B * G
ICH = 128
NCHUNK = TOK // ICH
CPW = NCHUNK // NW
KIN = 8
NOUT = CPW // KIN


@functools.lru_cache(maxsize=1)
def _build_sc_gather():
    @functools.partial(
        pl.kernel,
        mesh=plsc.VectorSubcoreMesh(core_axis_name="c", subcore_axis_name="s"),
        out_type=jax.ShapeDtypeStruct((NCHUNK, ICH, D), jnp.float32),
        scratch_types=[
            pltpu.VMEM((KIN, ICH), jnp.int32),
            pltpu.VMEM((KIN, ICH, D), jnp.float32),
            pltpu.SemaphoreType.DMA,
        ],
        compiler_params=pltpu.CompilerParams(use_tc_tiling_on_sc=False),
    )
    def _sc_gather(fidx_hbm, table_hbm, out_hbm, idx_v, emb_v, sem):
        wid = lax.axis_index("s") * NC + lax.axis_index("c")
        base = wid * CPW

        def step(t, carry):
            cb = base + t * KIN
            pltpu.sync_copy(fidx_hbm.at[pl.ds(cb, KIN)], idx_v)
            copies = [
                pltpu.async_copy(table_hbm.at[idx_v.at[j]], emb_v.at[j], sem)
                for j in range(KIN)
            ]
            for cp in copies:
                cp.wait()
            pltpu.sync_copy(emb_v, out_hbm.at[pl.ds(cb, KIN)])
            return carry

        lax.fori_loop(0, NOUT, step, 0)

    return _sc_gather


def kernel(x, tables):
    x3 = x.reshape(B, G, C)
    fidx = _tc_argmax(x3)
    fidx2 = fidx.reshape(NCHUNK, ICH)
    table2 = tables.reshape(G * C, D)
    out3 = _build_sc_gather()(fidx2, table2)
    return out3.reshape(B, G * D)


# speedup vs baseline: 8.0808x; 1.1954x over previous
import functools

import jax
import jax.numpy as jnp
from jax import lax
from jax.experimental import pallas as pl
from jax.experimental.pallas import tpu as pltpu
from jax.experimental.pallas import tpu_sc as plsc

B, G, C, D = 16384, 26, 100, 32

BB = 256


def _argmax_body(x_ref, o_ref):
    xv = x_ref[...]
    idx = jnp.argmax(xv, axis=-1)
    goff = lax.broadcasted_iota(jnp.int32, idx.shape, 1) * C
    o_ref[...] = idx.astype(jnp.int32) + goff


def _tc_argmax(x3):
    return pl.pallas_call(
        _argmax_body,
        grid=(B // BB,),
        in_specs=[pl.BlockSpec((BB, G, C), lambda i: (i, 0, 0))],
        out_specs=pl.BlockSpec((BB, G), lambda i: (i, 0)),
        out_shape=jax.ShapeDtypeStruct((B, G), jnp.int32),
    )(x3)


NC, NS = 2, 16
NW = NC * NS
TOK = B * G
ICH = 128
NCHUNK = TOK // ICH
CPW = NCHUNK // NW
KIN = 8
NOUT = CPW // KIN


@functools.lru_cache(maxsize=1)
def _build_sc_gather():
    @functools.partial(
        pl.kernel,
        mesh=plsc.VectorSubcoreMesh(core_axis_name="c", subcore_axis_name="s"),
        out_type=jax.ShapeDtypeStruct((NCHUNK, ICH, D), jnp.float32),
        scratch_types=[
            pltpu.VMEM((KIN, ICH), jnp.int32),
            pltpu.VMEM((KIN, ICH, D), jnp.float32),
            pltpu.SemaphoreType.DMA,
        ],
        compiler_params=pltpu.CompilerParams(use_tc_tiling_on_sc=False),
    )
    def _sc_gather(fidx_hbm, table_hbm, out_hbm, idx_v, emb_v, sem):
        wid = lax.axis_index("s") * NC + lax.axis_index("c")
        base = wid * CPW

        def step(t, carry):
            cb = base + t * KIN
            pltpu.sync_copy(fidx_hbm.at[pl.ds(cb, KIN)], idx_v)
            copies = [
                pltpu.async_copy(table_hbm.at[idx_v.at[j]], emb_v.at[j], sem)
                for j in range(KIN)
            ]
            for cp in copies:
                cp.wait()
            pltpu.sync_copy(emb_v, out_hbm.at[pl.ds(cb, KIN)])
            return carry

        lax.fori_loop(0, NOUT, step, 0)

    return _sc_gather


def kernel(x, tables):
    x3 = x.reshape(B, G, C)
    fidx = _tc_argmax(x3)
    fidx2 = fidx.reshape(NCHUNK, ICH)
    table2 = tables.reshape(G * C, D)
    out3 = _build_sc_gather()(fidx2, table2)
    return out3.reshape(B, G * D)


# speedup vs baseline: 10.1546x; 1.2566x over previous
import functools

import jax
import jax.numpy as jnp
from jax import lax
from jax.experimental import pallas as pl
from jax.experimental.pallas import tpu as pltpu
from jax.experimental.pallas import tpu_sc as plsc

B, G, C, D = 16384, 26, 100, 32

BB = 256


def _argmax_body(x_ref, o_ref):
    cols = []
    for g in range(G):
        xg = x_ref[:, g * C:(g + 1) * C]
        cols.append(jnp.argmax(xg, axis=-1).astype(jnp.int32) + g * C)
    o_ref[...] = jnp.stack(cols, axis=1)


def _tc_argmax(x):
    return pl.pallas_call(
        _argmax_body,
        grid=(B // BB,),
        in_specs=[pl.BlockSpec((BB, G * C), lambda i: (i, 0))],
        out_specs=pl.BlockSpec((BB, G), lambda i: (i, 0)),
        out_shape=jax.ShapeDtypeStruct((B, G), jnp.int32),
    )(x)


NC, NS = 2, 16
NW = NC * NS
TOK = B * G
ICH = 128
NCHUNK = TOK // ICH
CPW = NCHUNK // NW
KIN = 8
NOUT = CPW // KIN


@functools.lru_cache(maxsize=1)
def _build_sc_gather():
    @functools.partial(
        pl.kernel,
        mesh=plsc.VectorSubcoreMesh(core_axis_name="c", subcore_axis_name="s"),
        out_type=jax.ShapeDtypeStruct((NCHUNK, ICH, D), jnp.float32),
        scratch_types=[
            pltpu.VMEM((KIN, ICH), jnp.int32),
            pltpu.VMEM((KIN, ICH, D), jnp.float32),
            pltpu.SemaphoreType.DMA,
        ],
        compiler_params=pltpu.CompilerParams(use_tc_tiling_on_sc=False),
    )
    def _sc_gather(fidx_hbm, table_hbm, out_hbm, idx_v, emb_v, sem):
        wid = lax.axis_index("s") * NC + lax.axis_index("c")
        base = wid * CPW

        def step(t, carry):
            cb = base + t * KIN
            pltpu.sync_copy(fidx_hbm.at[pl.ds(cb, KIN)], idx_v)
            copies = [
                pltpu.async_copy(table_hbm.at[idx_v.at[j]], emb_v.at[j], sem)
                for j in range(KIN)
            ]
            for cp in copies:
                cp.wait()
            pltpu.sync_copy(emb_v, out_hbm.at[pl.ds(cb, KIN)])
            return carry

        lax.fori_loop(0, NOUT, step, 0)

    return _sc_gather


def kernel(x, tables):
    fidx = _tc_argmax(x)
    fidx2 = fidx.reshape(NCHUNK, ICH)
    table2 = tables.reshape(G * C, D)
    out3 = _build_sc_gather()(fidx2, table2)
    return out3.reshape(B, G * D)
